# Initial kernel scaffold; baseline (speedup 1.0000x reference)
import jax
import jax.numpy as jnp
from jax.experimental import pallas as pl


def kernel(x, edge_index, edge_attr, W_neigh1, b_neigh1, W_root1, gamma1, beta1, W_neigh2, b_neigh2, W_root2, gamma2, beta2, W_out, b_out):
    raise NotImplementedError("write your pallas kernel here")


# speedup vs baseline: 2.3456x; 2.3456x over previous
import functools

import jax
import jax.numpy as jnp
from jax import lax
from jax.experimental import pallas as pl
from jax.experimental.pallas import tpu as pltpu
from jax.experimental.pallas import tpu_sc as plsc

N = 10000
E = 320000
D = 128
DE = 16
H = 64
EPS = 1e-5

NC = 2
NS = 16
NW = NC * NS
B = 128
KSUB = 4
C = B * KSUB
EPW = 10240
E_PAD = EPW * NW
CHUNKS = EPW // C
IDXROWS = E_PAD // B
N_PAD = 10240
RPT = N_PAD // NS

_f32 = jnp.float32


def _make_sc_layer(with_deg: bool):
    mesh = plsc.VectorSubcoreMesh(core_axis_name="c", subcore_axis_name="s")
    out_type = [jax.ShapeDtypeStruct((NC, N_PAD, H), _f32)]
    if with_deg:
        out_type.append(jax.ShapeDtypeStruct((NC, N_PAD, 16), _f32))
    scratch_types = [
        pltpu.VMEM((KSUB, B), jnp.int32),
        pltpu.VMEM((KSUB, B), jnp.int32),
        pltpu.VMEM((C, H), _f32),
        pltpu.VMEM((C, H), _f32),
        pltpu.VMEM((B, 16), _f32),
        pltpu.VMEM_SHARED((N_PAD, H), _f32),
    ]
    if with_deg:
        scratch_types.append(pltpu.VMEM_SHARED((N_PAD, 16), _f32))
    scratch_types.append(pltpu.SemaphoreType.DMA)

    def body(table, eaw, srcm, dstm, *rest):
        if with_deg:
            (s_out, deg_out, idx_s, idx_d, rows_v, ea_v, ones_v, s_sh,
             deg_sh, sem) = rest
        else:
            (s_out, idx_s, idx_d, rows_v, ea_v, ones_v, s_sh, sem) = rest
            deg_out = deg_sh = None

        cid = lax.axis_index("c")
        sid = lax.axis_index("s")
        wid = sid * NC + cid
        zero16 = jnp.zeros((16,), _f32)

        def zrow(j, _):
            for k in range(H // 16):
                rows_v[j, pl.ds(k * 16, 16)] = zero16
            return 0
        lax.fori_loop(0, C, zrow, 0, unroll=4)

        def zrow16(j, _):
            ones_v[j, :] = zero16
            return 0
        lax.fori_loop(0, B, zrow16, 0, unroll=4)

        base_r = sid * RPT
        pltpu.sync_copy(rows_v, s_sh.at[pl.ds(base_r, C)])
        pltpu.sync_copy(rows_v.at[pl.ds(0, RPT - C)],
                        s_sh.at[pl.ds(base_r + C, RPT - C)])
        if with_deg:
            for t in range(RPT // B):
                pltpu.sync_copy(ones_v, deg_sh.at[pl.ds(base_r + t * B, B)])

        one16 = jnp.full((16,), 1.0, _f32)

        def orow(j, _):
            ones_v[j, :] = one16
            return 0
        lax.fori_loop(0, B, orow, 0, unroll=4)
        plsc.subcore_barrier()

        def step(c, _):
            row0 = wid * (EPW // B) + c * KSUB
            ebase = row0 * B
            pltpu.sync_copy(srcm.at[pl.ds(row0, KSUB)], idx_s)
            pltpu.sync_copy(dstm.at[pl.ds(row0, KSUB)], idx_d)
            pltpu.sync_copy(eaw.at[pl.ds(ebase, C)], ea_v)
            descs = []
            for k in range(KSUB):
                descs.append(pltpu.async_copy(
                    table.at[idx_s.at[k]],
                    rows_v.at[pl.ds(k * B, B)], sem))
            for dsc in descs:
                dsc.wait()

            def compute(j, _):
                for k in range(H // 16):
                    sl = pl.ds(k * 16, 16)
                    ea_v[j, sl] = jnp.maximum(rows_v[j, sl] + ea_v[j, sl], 0.0)
                return 0
            lax.fori_loop(0, C, compute, 0, unroll=2)

            for k in range(KSUB):
                pltpu.sync_copy(ea_v.at[pl.ds(k * B, B)],
                                s_sh.at[idx_d.at[k]], add=True)
            if with_deg:
                for k in range(KSUB):
                    pltpu.sync_copy(ones_v, deg_sh.at[idx_d.at[k]], add=True)
            return 0
        lax.fori_loop(0, CHUNKS, step, 0)
        plsc.subcore_barrier()

        pltpu.sync_copy(s_sh.at[pl.ds(base_r, C)], rows_v)
        pltpu.sync_copy(rows_v, s_out.at[cid, pl.ds(base_r, C)])
        pltpu.sync_copy(s_sh.at[pl.ds(base_r + C, RPT - C)],
                        rows_v.at[pl.ds(0, RPT - C)])
        pltpu.sync_copy(rows_v.at[pl.ds(0, RPT - C)],
                        s_out.at[cid, pl.ds(base_r + C, RPT - C)])
        if with_deg:
            for t in range(RPT // B):
                r = base_r + t * B
                pltpu.sync_copy(deg_sh.at[pl.ds(r, B)], ones_v)
                pltpu.sync_copy(ones_v, deg_out.at[cid, pl.ds(r, B)])

    return pl.kernel(body, out_type=out_type, mesh=mesh,
                     scratch_types=scratch_types,
                     compiler_params=pltpu.CompilerParams(
                         use_tc_tiling_on_sc=False))


_sc_layer_deg = _make_sc_layer(True)
_sc_layer = _make_sc_layer(False)


def _kx_body(x_ref, w_ref, o1_ref, o2_ref):
    y = jnp.dot(x_ref[...], w_ref[...], preferred_element_type=_f32)
    o1_ref[...] = y[:, :H]
    o2_ref[...] = y[:, H:]


def _kea_body(ea_ref, w_ref, b_ref, o1_ref, o2_ref):
    y = jnp.dot(ea_ref[...], w_ref[...], preferred_element_type=_f32)
    y = y + b_ref[...]
    o1_ref[...] = y[:, :H]
    o2_ref[...] = y[:, H:]


def _kmid_body(s_ref, deg_ref, xr_ref, g_ref, b_ref, w_ref, o1_ref, o2_ref):
    s = s_ref[0, :N, :] + s_ref[1, :N, :]
    deg = deg_ref[0, :N, 0:1] + deg_ref[1, :N, 0:1]
    h = xr_ref[...] + s / jnp.maximum(deg, 1.0)
    mu = jnp.mean(h, axis=0, keepdims=True)
    var = jnp.mean((h - mu) ** 2, axis=0, keepdims=True)
    h = (h - mu) * lax.rsqrt(var + EPS) * g_ref[...] + b_ref[...]
    h = jnp.maximum(h, 0.0)
    y = jnp.dot(h, w_ref[...], preferred_element_type=_f32)
    o1_ref[...] = y[:, :H]
    o2_ref[...] = y[:, H:]


def _kfin_body(s_ref, deg_ref, xr_ref, g_ref, b_ref, wo_ref, bo_ref, o_ref):
    s = s_ref[0, :N, :] + s_ref[1, :N, :]
    deg = deg_ref[0, :N, 0:1] + deg_ref[1, :N, 0:1]
    h = xr_ref[...] + s / jnp.maximum(deg, 1.0)
    mu = jnp.mean(h, axis=0, keepdims=True)
    var = jnp.mean((h - mu) ** 2, axis=0, keepdims=True)
    h = (h - mu) * lax.rsqrt(var + EPS) * g_ref[...] + b_ref[...]
    h = jnp.maximum(h, 0.0)
    hg = jnp.mean(h, axis=0, keepdims=True)
    o_ref[...] = jnp.dot(hg, wo_ref[...], preferred_element_type=_f32) + bo_ref[...]


_EA_BLK = 4096


def _run_tc(x, Wcat1, ea_p, Wecat, bcat):
    xW1, xroot1 = pl.pallas_call(
        _kx_body,
        out_shape=[jax.ShapeDtypeStruct((N, H), _f32),
                   jax.ShapeDtypeStruct((N, H), _f32)],
    )(x, Wcat1)
    eaW1, eaW2 = pl.pallas_call(
        _kea_body,
        grid=(E_PAD // _EA_BLK,),
        in_specs=[pl.BlockSpec((_EA_BLK, DE), lambda i: (i, 0)),
                  pl.BlockSpec((DE, 2 * H), lambda i: (0, 0)),
                  pl.BlockSpec((1, 2 * H), lambda i: (0, 0))],
        out_specs=[pl.BlockSpec((_EA_BLK, H), lambda i: (i, 0)),
                   pl.BlockSpec((_EA_BLK, H), lambda i: (i, 0))],
        out_shape=[jax.ShapeDtypeStruct((E_PAD, H), _f32),
                   jax.ShapeDtypeStruct((E_PAD, H), _f32)],
    )(ea_p, Wecat, bcat)
    return xW1, xroot1, eaW1, eaW2


def kernel(x, edge_index, edge_attr, W_neigh1, b_neigh1, W_root1, gamma1,
           beta1, W_neigh2, b_neigh2, W_root2, gamma2, beta2, W_out, b_out):
    src = edge_index[0].astype(jnp.int32)
    dst = edge_index[1].astype(jnp.int32)
    pad = E_PAD - E
    srcm = jnp.concatenate([src, jnp.zeros((pad,), jnp.int32)]).reshape(IDXROWS, B)
    dstm = jnp.concatenate([dst, jnp.full((pad,), N, jnp.int32)]).reshape(IDXROWS, B)
    ea_p = jnp.concatenate([edge_attr, jnp.zeros((pad, DE), _f32)], axis=0)

    Wcat1 = jnp.concatenate([W_neigh1[:D], W_root1], axis=1)
    Wecat = jnp.concatenate([W_neigh1[D:], W_neigh2[H:]], axis=1)
    bcat = jnp.concatenate([b_neigh1, b_neigh2]).reshape(1, 2 * H)
    Wcat2 = jnp.concatenate([W_neigh2[:H], W_root2], axis=1)

    xW1, xroot1, eaW1, eaW2 = _run_tc(x, Wcat1, ea_p, Wecat, bcat)

    s1p, degp = _sc_layer_deg(xW1, eaW1, srcm, dstm)

    xW2, hroot2 = pl.pallas_call(
        _kmid_body,
        out_shape=[jax.ShapeDtypeStruct((N, H), _f32),
                   jax.ShapeDtypeStruct((N, H), _f32)],
    )(s1p, degp, xroot1, gamma1.reshape(1, H), beta1.reshape(1, H), Wcat2)

    (s2p,) = _sc_layer(xW2, eaW2, srcm, dstm)

    out = pl.pallas_call(
        _kfin_body,
        out_shape=jax.ShapeDtypeStruct((1, 1), _f32),
    )(s2p, degp, hroot2, gamma2.reshape(1, H), beta2.reshape(1, H),
      W_out, b_out.reshape(1, 1))
    return out.reshape(1)


# speedup vs baseline: 3.2860x; 1.4009x over previous
import functools

import jax
import jax.numpy as jnp
from jax import lax
from jax.experimental import pallas as pl
from jax.experimental.pallas import tpu as pltpu
from jax.experimental.pallas import tpu_sc as plsc

N = 10000
E = 320000
D = 128
DE = 16
H = 64
EPS = 1e-5

NC = 2
NS = 16
NW = NC * NS
B = 128
KSUB = 2
C = B * KSUB
EPW = 10240
E_PAD = EPW * NW
CHUNKS = EPW // C
ROWS_PW = EPW // B
IDXROWS = E_PAD // B
N_PAD = 10240
RPT = N_PAD // NS

_f32 = jnp.float32


_SC_MESH = plsc.VectorSubcoreMesh(core_axis_name="c", subcore_axis_name="s")
_SC_PARAMS = pltpu.CompilerParams(use_tc_tiling_on_sc=False)


def _make_sc_layer():
    out_type = [jax.ShapeDtypeStruct((NC, N_PAD, H), _f32)]
    scratch_types = [
        pltpu.VMEM((ROWS_PW, B), jnp.int32),
        pltpu.VMEM((ROWS_PW, B), jnp.int32),
        pltpu.VMEM((C, H), _f32),
        pltpu.VMEM((C, H), _f32),
        pltpu.VMEM((C, H), _f32),
        pltpu.VMEM((C, H), _f32),
        pltpu.VMEM_SHARED((N_PAD, H), _f32),
    ]
    scratch_types += [pltpu.SemaphoreType.DMA] * 4

    def body(table, eaw, srcm, dstm, s_out, idx_s, idx_d, rows0, rows1,
             ea0, ea1, s_sh, sg0, sg1, se0, se1):
        rows = (rows0, rows1)
        ea = (ea0, ea1)
        sg = (sg0, sg1)
        se = (se0, se1)

        cid = lax.axis_index("c")
        sid = lax.axis_index("s")
        wid = sid * NC + cid
        row_base = wid * ROWS_PW
        zero16 = jnp.zeros((16,), _f32)

        pltpu.sync_copy(srcm.at[pl.ds(row_base, ROWS_PW)], idx_s)
        pltpu.sync_copy(dstm.at[pl.ds(row_base, ROWS_PW)], idx_d)

        def zrow(j, _):
            for k in range(H // 16):
                rows0[j, pl.ds(k * 16, 16)] = zero16
            return 0
        lax.fori_loop(0, C, zrow, 0, unroll=4)

        base_r = sid * RPT
        for r0 in range(0, RPT, C):
            w = min(C, RPT - r0)
            pltpu.sync_copy(rows0.at[pl.ds(0, w)],
                            s_sh.at[pl.ds(base_r + r0, w)])
        plsc.subcore_barrier()

        def issue(i, p):
            lrow = i * KSUB
            pltpu.async_copy(eaw.at[pl.ds((row_base + lrow) * B, C)],
                             ea[p], se[p])
            for k in range(KSUB):
                pltpu.async_copy(table.at[idx_s.at[lrow + k]],
                                 rows[p].at[pl.ds(k * B, B)], sg[p])

        def consume(i, p):
            pltpu.make_async_copy(table.at[pl.ds(0, C)], rows[p], sg[p]).wait()
            pltpu.make_async_copy(eaw.at[pl.ds(0, C)], ea[p], se[p]).wait()

            def compute(j, _):
                for k in range(H // 16):
                    sl = pl.ds(k * 16, 16)
                    ea[p][j, sl] = jnp.maximum(
                        rows[p][j, sl] + ea[p][j, sl], 0.0)
                return 0
            lax.fori_loop(0, C, compute, 0, unroll=2)

            lrow = i * KSUB
            for k in range(KSUB):
                pltpu.sync_copy(ea[p].at[pl.ds(k * B, B)],
                                s_sh.at[idx_d.at[lrow + k]], add=True)

        issue(0, 0)

        def step(t, _):
            i = 2 * t
            issue(i + 1, 1)
            consume(i, 0)
            issue(i + 2, 0)
            consume(i + 1, 1)
            return 0
        lax.fori_loop(0, CHUNKS // 2 - 1, step, 0)
        issue(CHUNKS - 1, 1)
        consume(CHUNKS - 2, 0)
        consume(CHUNKS - 1, 1)
        plsc.subcore_barrier()

        for r0 in range(0, RPT, C):
            w = min(C, RPT - r0)
            pltpu.sync_copy(s_sh.at[pl.ds(base_r + r0, w)],
                            rows0.at[pl.ds(0, w)])
            pltpu.sync_copy(rows0.at[pl.ds(0, w)],
                            s_out.at[cid, pl.ds(base_r + r0, w)])

    return pl.kernel(body, out_type=out_type, mesh=_SC_MESH,
                     scratch_types=scratch_types,
                     compiler_params=_SC_PARAMS)


def _make_sc_deg():
    out_type = [jax.ShapeDtypeStruct((NC, N_PAD, 16), _f32)]
    scratch_types = [
        pltpu.VMEM((ROWS_PW, B), jnp.int32),
        pltpu.VMEM((B, 16), _f32),
        pltpu.VMEM_SHARED((N_PAD, 16), _f32),
    ]

    def body(dstm, deg_out, idx_d, ones_v, deg_sh):
        cid = lax.axis_index("c")
        sid = lax.axis_index("s")
        wid = sid * NC + cid
        zero16 = jnp.zeros((16,), _f32)
        one16 = jnp.full((16,), 1.0, _f32)

        pltpu.sync_copy(dstm.at[pl.ds(wid * ROWS_PW, ROWS_PW)], idx_d)

        def zrow16(j, _):
            ones_v[j, :] = zero16
            return 0
        lax.fori_loop(0, B, zrow16, 0, unroll=4)
        base_r = sid * RPT
        for t in range(RPT // B):
            pltpu.sync_copy(ones_v, deg_sh.at[pl.ds(base_r + t * B, B)])

        def orow(j, _):
            ones_v[j, :] = one16
            return 0
        lax.fori_loop(0, B, orow, 0, unroll=4)
        plsc.subcore_barrier()

        def step(r, _):
            pltpu.sync_copy(ones_v, deg_sh.at[idx_d.at[r]], add=True)
            return 0
        lax.fori_loop(0, ROWS_PW, step, 0)
        plsc.subcore_barrier()

        for t in range(RPT // B):
            r = base_r + t * B
            pltpu.sync_copy(deg_sh.at[pl.ds(r, B)], ones_v)
            pltpu.sync_copy(ones_v, deg_out.at[cid, pl.ds(r, B)])

    return pl.kernel(body, out_type=out_type, mesh=_SC_MESH,
                     scratch_types=scratch_types,
                     compiler_params=_SC_PARAMS)


_sc_layer = _make_sc_layer()
_sc_deg = _make_sc_deg()


def _dot(a, b):
    return jnp.dot(a, b, preferred_element_type=_f32,
                   precision=lax.Precision.HIGHEST)


def _kx_body(x_ref, w_ref, o1_ref, o2_ref):
    y = _dot(x_ref[...], w_ref[...])
    o1_ref[...] = y[:, :H]
    o2_ref[...] = y[:, H:]


def _kea_body(ea_ref, w_ref, b_ref, o1_ref, o2_ref):
    y = _dot(ea_ref[...], w_ref[...])
    y = y + b_ref[...]
    o1_ref[...] = y[:, :H]
    o2_ref[...] = y[:, H:]


def _kmid_body(s_ref, deg_ref, xr_ref, g_ref, b_ref, w_ref, o1_ref, o2_ref):
    s = s_ref[0, :N, :] + s_ref[1, :N, :]
    deg = deg_ref[0, :N, 0:1] + deg_ref[1, :N, 0:1]
    h = xr_ref[...] + s / jnp.maximum(deg, 1.0)
    mu = jnp.mean(h, axis=0, keepdims=True)
    var = jnp.mean((h - mu) ** 2, axis=0, keepdims=True)
    h = (h - mu) * lax.rsqrt(var + EPS) * g_ref[...] + b_ref[...]
    h = jnp.maximum(h, 0.0)
    y = _dot(h, w_ref[...])
    o1_ref[...] = y[:, :H]
    o2_ref[...] = y[:, H:]


def _kfin_body(s_ref, deg_ref, xr_ref, g_ref, b_ref, wo_ref, bo_ref, o_ref):
    s = s_ref[0, :N, :] + s_ref[1, :N, :]
    deg = deg_ref[0, :N, 0:1] + deg_ref[1, :N, 0:1]
    h = xr_ref[...] + s / jnp.maximum(deg, 1.0)
    mu = jnp.mean(h, axis=0, keepdims=True)
    var = jnp.mean((h - mu) ** 2, axis=0, keepdims=True)
    h = (h - mu) * lax.rsqrt(var + EPS) * g_ref[...] + b_ref[...]
    h = jnp.maximum(h, 0.0)
    hg = jnp.mean(h, axis=0, keepdims=True)
    o_ref[...] = _dot(hg, wo_ref[...]) + bo_ref[...]


_EA_BLK = 4096


def _run_tc(x, Wcat1, ea_p, Wecat, bcat):
    xW1, xroot1 = pl.pallas_call(
        _kx_body,
        out_shape=[jax.ShapeDtypeStruct((N, H), _f32),
                   jax.ShapeDtypeStruct((N, H), _f32)],
    )(x, Wcat1)
    eaW1, eaW2 = pl.pallas_call(
        _kea_body,
        grid=(E_PAD // _EA_BLK,),
        in_specs=[pl.BlockSpec((_EA_BLK, DE), lambda i: (i, 0)),
                  pl.BlockSpec((DE, 2 * H), lambda i: (0, 0)),
                  pl.BlockSpec((1, 2 * H), lambda i: (0, 0))],
        out_specs=[pl.BlockSpec((_EA_BLK, H), lambda i: (i, 0)),
                   pl.BlockSpec((_EA_BLK, H), lambda i: (i, 0))],
        out_shape=[jax.ShapeDtypeStruct((E_PAD, H), _f32),
                   jax.ShapeDtypeStruct((E_PAD, H), _f32)],
    )(ea_p, Wecat, bcat)
    return xW1, xroot1, eaW1, eaW2


def kernel(x, edge_index, edge_attr, W_neigh1, b_neigh1, W_root1, gamma1,
           beta1, W_neigh2, b_neigh2, W_root2, gamma2, beta2, W_out, b_out):
    src = edge_index[0].astype(jnp.int32)
    dst = edge_index[1].astype(jnp.int32)
    pad = E_PAD - E
    srcm = jnp.concatenate([src, jnp.zeros((pad,), jnp.int32)]).reshape(IDXROWS, B)
    dstm = jnp.concatenate([dst, jnp.full((pad,), N, jnp.int32)]).reshape(IDXROWS, B)
    ea_p = jnp.concatenate([edge_attr, jnp.zeros((pad, DE), _f32)], axis=0)

    Wcat1 = jnp.concatenate([W_neigh1[:D], W_root1], axis=1)
    Wecat = jnp.concatenate([W_neigh1[D:], W_neigh2[H:]], axis=1)
    bcat = jnp.concatenate([b_neigh1, b_neigh2]).reshape(1, 2 * H)
    Wcat2 = jnp.concatenate([W_neigh2[:H], W_root2], axis=1)

    xW1, xroot1, eaW1, eaW2 = _run_tc(x, Wcat1, ea_p, Wecat, bcat)

    (degp,) = _sc_deg(dstm)
    (s1p,) = _sc_layer(xW1, eaW1, srcm, dstm)

    xW2, hroot2 = pl.pallas_call(
        _kmid_body,
        out_shape=[jax.ShapeDtypeStruct((N, H), _f32),
                   jax.ShapeDtypeStruct((N, H), _f32)],
    )(s1p, degp, xroot1, gamma1.reshape(1, H), beta1.reshape(1, H), Wcat2)

    (s2p,) = _sc_layer(xW2, eaW2, srcm, dstm)

    out = pl.pallas_call(
        _kfin_body,
        out_shape=jax.ShapeDtypeStruct((1, 1), _f32),
    )(s2p, degp, hroot2, gamma2.reshape(1, H), beta2.reshape(1, H),
      W_out, b_out.reshape(1, 1))
    return out.reshape(1)


# speedup vs baseline: 4.0428x; 1.2303x over previous
import jax
import jax.numpy as jnp
from jax import lax
from jax.experimental import pallas as pl
from jax.experimental.pallas import tpu as pltpu
from jax.experimental.pallas import tpu_sc as plsc

N = 10000
E = 320000
D = 128
DE = 16
H = 64
EPS = 1e-5

NC = 2
NS = 16
NW = NC * NS
B = 128
KSUB = 4
C = B * KSUB
EPW = 10240
E_PAD = EPW * NW
CHUNKS = EPW // C
ROWS_PW = EPW // B
IDXROWS = E_PAD // B
N_PAD = 10240
RPT = N_PAD // NS

_f32 = jnp.float32
_bf16 = jnp.bfloat16


_SC_MESH = plsc.VectorSubcoreMesh(core_axis_name="c", subcore_axis_name="s")
_SC_PARAMS = pltpu.CompilerParams(use_tc_tiling_on_sc=False)


def _make_sc_layer(with_deg: bool):
    out_type = [jax.ShapeDtypeStruct((NC, N_PAD, H), _bf16)]
    if with_deg:
        out_type.append(jax.ShapeDtypeStruct((NC, N_PAD, 16), _bf16))
    scratch_types = [
        pltpu.VMEM((ROWS_PW, B), jnp.int32),
        pltpu.VMEM((ROWS_PW, B), jnp.int32),
        pltpu.VMEM((C, H), _bf16),
        pltpu.VMEM((C, H), _bf16),
        pltpu.VMEM((C, H), _bf16),
        pltpu.VMEM((C, H), _bf16),
        pltpu.VMEM_SHARED((N_PAD, H), _bf16),
    ]
    if with_deg:
        scratch_types.append(pltpu.VMEM((B, 16), _bf16))
        scratch_types.append(pltpu.VMEM_SHARED((N_PAD, 16), _bf16))
    scratch_types += [pltpu.SemaphoreType.DMA] * 4

    def body(table, eaw, srcm, dstm, *rest):
        if with_deg:
            (s_out, deg_out, idx_s, idx_d, rows0, rows1, ea0, ea1, s_sh,
             ones_v, deg_sh, sg0, sg1, se0, se1) = rest
        else:
            (s_out, idx_s, idx_d, rows0, rows1, ea0, ea1, s_sh,
             sg0, sg1, se0, se1) = rest
            deg_out = ones_v = deg_sh = None
        rows = (rows0, rows1)
        ea = (ea0, ea1)
        sg = (sg0, sg1)
        se = (se0, se1)

        cid = lax.axis_index("c")
        sid = lax.axis_index("s")
        wid = sid * NC + cid
        row_base = wid * ROWS_PW
        zero32 = jnp.zeros((32,), _bf16)

        pltpu.sync_copy(srcm.at[pl.ds(row_base, ROWS_PW)], idx_s)
        pltpu.sync_copy(dstm.at[pl.ds(row_base, ROWS_PW)], idx_d)

        def zrow(j, _):
            for k in range(H // 32):
                rows0[j, pl.ds(k * 32, 32)] = zero32
            return 0
        lax.fori_loop(0, C, zrow, 0, unroll=4)

        base_r = sid * RPT
        for r0 in range(0, RPT, C):
            w = min(C, RPT - r0)
            pltpu.sync_copy(rows0.at[pl.ds(0, w)],
                            s_sh.at[pl.ds(base_r + r0, w)])
        if with_deg:
            z216 = jnp.zeros((2, 16), _bf16)

            def zo(j, _):
                ones_v[pl.ds(2 * j, 2), :] = z216
                return 0
            lax.fori_loop(0, B // 2, zo, 0, unroll=4)
            for t in range(RPT // B):
                pltpu.sync_copy(ones_v, deg_sh.at[pl.ds(base_r + t * B, B)])
            o216 = jnp.full((2, 16), 1.0, _bf16)

            def oo(j, _):
                ones_v[pl.ds(2 * j, 2), :] = o216
                return 0
            lax.fori_loop(0, B // 2, oo, 0, unroll=4)

        plsc.subcore_barrier()

        def issue(i, p):
            lrow = i * KSUB
            pltpu.async_copy(eaw.at[pl.ds((row_base + lrow) * B, C)],
                             ea[p], se[p])
            for k in range(KSUB):
                pltpu.async_copy(table.at[idx_s.at[lrow + k]],
                                 rows[p].at[pl.ds(k * B, B)], sg[p])

        def consume(i, p):
            pltpu.make_async_copy(table.at[pl.ds(0, C)], rows[p], sg[p]).wait()
            pltpu.make_async_copy(eaw.at[pl.ds(0, C)], ea[p], se[p]).wait()

            def compute(j, _):
                for k in range(H // 32):
                    sl = pl.ds(k * 32, 32)
                    ea[p][j, sl] = jnp.maximum(
                        rows[p][j, sl] + ea[p][j, sl], 0.0)
                return 0
            lax.fori_loop(0, C, compute, 0, unroll=2)

            lrow = i * KSUB
            for k in range(KSUB):
                pltpu.sync_copy(ea[p].at[pl.ds(k * B, B)],
                                s_sh.at[idx_d.at[lrow + k]], add=True)
            if with_deg:
                for k in range(KSUB):
                    pltpu.sync_copy(ones_v, deg_sh.at[idx_d.at[lrow + k]],
                                    add=True)

        issue(0, 0)

        def step(t, _):
            i = 2 * t
            issue(i + 1, 1)
            consume(i, 0)
            issue(i + 2, 0)
            consume(i + 1, 1)
            return 0
        lax.fori_loop(0, CHUNKS // 2 - 1, step, 0)
        issue(CHUNKS - 1, 1)
        consume(CHUNKS - 2, 0)
        consume(CHUNKS - 1, 1)
        plsc.subcore_barrier()

        for r0 in range(0, RPT, C):
            w = min(C, RPT - r0)
            pltpu.sync_copy(s_sh.at[pl.ds(base_r + r0, w)],
                            rows0.at[pl.ds(0, w)])
            pltpu.sync_copy(rows0.at[pl.ds(0, w)],
                            s_out.at[cid, pl.ds(base_r + r0, w)])
        if with_deg:
            for t in range(RPT // B):
                r = base_r + t * B
                pltpu.sync_copy(deg_sh.at[pl.ds(r, B)], ones_v)
                pltpu.sync_copy(ones_v, deg_out.at[cid, pl.ds(r, B)])

    return pl.kernel(body, out_type=out_type, mesh=_SC_MESH,
                     scratch_types=scratch_types,
                     compiler_params=_SC_PARAMS)


_sc_layer_deg = _make_sc_layer(True)
_sc_layer = _make_sc_layer(False)


def _dot(a, b):
    return jnp.dot(a, b, preferred_element_type=_f32,
                   precision=lax.Precision.HIGHEST)


def _kx_body(x_ref, w_ref, o1_ref, o2_ref):
    y = _dot(x_ref[...], w_ref[...])
    o1_ref[...] = y[:, :H].astype(_bf16)
    o2_ref[...] = y[:, H:]


def _kea_body(ea_ref, w_ref, b_ref, o1_ref, o2_ref):
    y = _dot(ea_ref[...], w_ref[...])
    y = y + b_ref[...]
    o1_ref[...] = y[:, :H].astype(_bf16)
    o2_ref[...] = y[:, H:].astype(_bf16)


def _kmid_body(s_ref, deg_ref, xr_ref, g_ref, b_ref, w_ref, o1_ref, o2_ref):
    s = s_ref[0, :N, :].astype(_f32) + s_ref[1, :N, :].astype(_f32)
    deg = deg_ref[0, :N, 0:1].astype(_f32) + deg_ref[1, :N, 0:1].astype(_f32)
    h = xr_ref[...] + s / jnp.maximum(deg, 1.0)
    mu = jnp.mean(h, axis=0, keepdims=True)
    var = jnp.mean((h - mu) ** 2, axis=0, keepdims=True)
    h = (h - mu) * lax.rsqrt(var + EPS) * g_ref[...] + b_ref[...]
    h = jnp.maximum(h, 0.0)
    y = _dot(h, w_ref[...])
    o1_ref[...] = y[:, :H].astype(_bf16)
    o2_ref[...] = y[:, H:]


def _kfin_body(s_ref, deg_ref, xr_ref, g_ref, b_ref, wo_ref, bo_ref, o_ref):
    s = s_ref[0, :N, :].astype(_f32) + s_ref[1, :N, :].astype(_f32)
    deg = deg_ref[0, :N, 0:1].astype(_f32) + deg_ref[1, :N, 0:1].astype(_f32)
    h = xr_ref[...] + s / jnp.maximum(deg, 1.0)
    mu = jnp.mean(h, axis=0, keepdims=True)
    var = jnp.mean((h - mu) ** 2, axis=0, keepdims=True)
    h = (h - mu) * lax.rsqrt(var + EPS) * g_ref[...] + b_ref[...]
    h = jnp.maximum(h, 0.0)
    hg = jnp.mean(h, axis=0, keepdims=True)
    o_ref[...] = _dot(hg, wo_ref[...]) + bo_ref[...]


_EA_BLK = 4096


def _run_tc(x, Wcat1, ea_p, Wecat, bcat):
    xW1, xroot1 = pl.pallas_call(
        _kx_body,
        out_shape=[jax.ShapeDtypeStruct((N, H), _bf16),
                   jax.ShapeDtypeStruct((N, H), _f32)],
    )(x, Wcat1)
    eaW1, eaW2 = pl.pallas_call(
        _kea_body,
        grid=(E_PAD // _EA_BLK,),
        in_specs=[pl.BlockSpec((_EA_BLK, DE), lambda i: (i, 0)),
                  pl.BlockSpec((DE, 2 * H), lambda i: (0, 0)),
                  pl.BlockSpec((1, 2 * H), lambda i: (0, 0))],
        out_specs=[pl.BlockSpec((_EA_BLK, H), lambda i: (i, 0)),
                   pl.BlockSpec((_EA_BLK, H), lambda i: (i, 0))],
        out_shape=[jax.ShapeDtypeStruct((E_PAD, H), _bf16),
                   jax.ShapeDtypeStruct((E_PAD, H), _bf16)],
    )(ea_p, Wecat, bcat)
    return xW1, xroot1, eaW1, eaW2


def kernel(x, edge_index, edge_attr, W_neigh1, b_neigh1, W_root1, gamma1,
           beta1, W_neigh2, b_neigh2, W_root2, gamma2, beta2, W_out, b_out):
    src = edge_index[0].astype(jnp.int32)
    dst = edge_index[1].astype(jnp.int32)
    pad = E_PAD - E
    srcm = jnp.concatenate([src, jnp.zeros((pad,), jnp.int32)]).reshape(IDXROWS, B)
    dstm = jnp.concatenate([dst, jnp.full((pad,), N, jnp.int32)]).reshape(IDXROWS, B)
    ea_p = jnp.concatenate([edge_attr, jnp.zeros((pad, DE), _f32)], axis=0)

    Wcat1 = jnp.concatenate([W_neigh1[:D], W_root1], axis=1)
    Wecat = jnp.concatenate([W_neigh1[D:], W_neigh2[H:]], axis=1)
    bcat = jnp.concatenate([b_neigh1, b_neigh2]).reshape(1, 2 * H)
    Wcat2 = jnp.concatenate([W_neigh2[:H], W_root2], axis=1)

    xW1, xroot1, eaW1, eaW2 = _run_tc(x, Wcat1, ea_p, Wecat, bcat)

    s1p, degp = _sc_layer_deg(xW1, eaW1, srcm, dstm)

    xW2, hroot2 = pl.pallas_call(
        _kmid_body,
        out_shape=[jax.ShapeDtypeStruct((N, H), _bf16),
                   jax.ShapeDtypeStruct((N, H), _f32)],
    )(s1p, degp, xroot1, gamma1.reshape(1, H), beta1.reshape(1, H), Wcat2)

    (s2p,) = _sc_layer(xW2, eaW2, srcm, dstm)

    out = pl.pallas_call(
        _kfin_body,
        out_shape=jax.ShapeDtypeStruct((1, 1), _f32),
    )(s2p, degp, hroot2, gamma2.reshape(1, H), beta2.reshape(1, H),
      W_out, b_out.reshape(1, 1))
    return out.reshape(1)


# speedup vs baseline: 4.4790x; 1.1079x over previous
import jax
import jax.numpy as jnp
from jax import lax
from jax.experimental import pallas as pl
from jax.experimental.pallas import tpu as pltpu
from jax.experimental.pallas import tpu_sc as plsc

N = 10000
E = 320000
D = 128
DE = 16
H = 64
EPS = 1e-5

NC = 2
NS = 16
NW = NC * NS
B = 128
NROWS = E // B
RPW_MAX = NROWS // NW + 1
PAIRS = NROWS // NW // 2
N_PAD = 10240
RPT = N_PAD // NS

_f32 = jnp.float32
_bf16 = jnp.bfloat16


_SC_MESH = plsc.VectorSubcoreMesh(core_axis_name="c", subcore_axis_name="s")
_SC_PARAMS = pltpu.CompilerParams(use_tc_tiling_on_sc=False)


def _make_sc_layer(with_deg: bool):
    out_type = [jax.ShapeDtypeStruct((NC, N_PAD, H), _bf16)]
    if with_deg:
        out_type.append(jax.ShapeDtypeStruct((NC, N_PAD, 16), _bf16))
    scratch_types = [
        pltpu.VMEM((RPW_MAX, B), jnp.int32),
        pltpu.VMEM((RPW_MAX, B), jnp.int32),
        pltpu.VMEM((B, H), _bf16),
        pltpu.VMEM((B, H), _bf16),
        pltpu.VMEM((B, H), _bf16),
        pltpu.VMEM((B, H), _bf16),
        pltpu.VMEM_SHARED((N_PAD, H), _bf16),
    ]
    if with_deg:
        scratch_types.append(pltpu.VMEM((B, 16), _bf16))
        scratch_types.append(pltpu.VMEM_SHARED((N_PAD, 16), _bf16))
    scratch_types += [pltpu.SemaphoreType.DMA] * 4

    def body(table, eaw, srcm, dstm, *rest):
        if with_deg:
            (s_out, deg_out, idx_s, idx_d, rows0, rows1, ea0, ea1, s_sh,
             ones_v, deg_sh, sg0, sg1, se0, se1) = rest
        else:
            (s_out, idx_s, idx_d, rows0, rows1, ea0, ea1, s_sh,
             sg0, sg1, se0, se1) = rest
            deg_out = ones_v = deg_sh = None
        rows = (rows0, rows1)
        ea = (ea0, ea1)
        sg = (sg0, sg1)
        se = (se0, se1)

        cid = lax.axis_index("c")
        sid = lax.axis_index("s")
        wid = sid * NC + cid
        lo = wid * NROWS // NW
        nr = (wid + 1) * NROWS // NW - lo
        zero32 = jnp.zeros((32,), _bf16)

        pltpu.sync_copy(srcm.at[pl.ds(lo, RPW_MAX)], idx_s)
        pltpu.sync_copy(dstm.at[pl.ds(lo, RPW_MAX)], idx_d)

        def zrow(j, _):
            for k in range(H // 32):
                rows0[j, pl.ds(k * 32, 32)] = zero32
            return 0
        lax.fori_loop(0, B, zrow, 0, unroll=4)

        base_r = sid * RPT
        for r0 in range(0, RPT, B):
            pltpu.sync_copy(rows0, s_sh.at[pl.ds(base_r + r0, B)])
        if with_deg:
            z216 = jnp.zeros((2, 16), _bf16)

            def zo(j, _):
                ones_v[pl.ds(2 * j, 2), :] = z216
                return 0
            lax.fori_loop(0, B // 2, zo, 0, unroll=4)
            for t in range(RPT // B):
                pltpu.sync_copy(ones_v, deg_sh.at[pl.ds(base_r + t * B, B)])
            o216 = jnp.full((2, 16), 1.0, _bf16)

            def oo(j, _):
                ones_v[pl.ds(2 * j, 2), :] = o216
                return 0
            lax.fori_loop(0, B // 2, oo, 0, unroll=4)

        plsc.subcore_barrier()

        def issue(r, p):
            pltpu.async_copy(eaw.at[pl.ds((lo + r) * B, B)], ea[p], se[p])
            pltpu.async_copy(table.at[idx_s.at[r]], rows[p], sg[p])

        def consume(r, p):
            pltpu.make_async_copy(table.at[pl.ds(0, B)], rows[p], sg[p]).wait()
            pltpu.make_async_copy(eaw.at[pl.ds(0, B)], ea[p], se[p]).wait()

            def compute(j, _):
                for k in range(H // 32):
                    sl = pl.ds(k * 32, 32)
                    ea[p][j, sl] = jnp.maximum(
                        rows[p][j, sl] + ea[p][j, sl], 0.0)
                return 0
            lax.fori_loop(0, B, compute, 0, unroll=2)

            pltpu.sync_copy(ea[p], s_sh.at[idx_d.at[r]], add=True)
            if with_deg:
                pltpu.sync_copy(ones_v, deg_sh.at[idx_d.at[r]], add=True)

        issue(0, 0)

        def step(t, _):
            r = 2 * t
            issue(r + 1, 1)
            consume(r, 0)

            @pl.when(r + 2 < nr)
            def _():
                issue(r + 2, 0)
            consume(r + 1, 1)
            return 0
        lax.fori_loop(0, PAIRS, step, 0)

        @pl.when(nr % 2 == 1)
        def _():
            consume(nr - 1, 0)
        plsc.subcore_barrier()

        for r0 in range(0, RPT, B):
            pltpu.sync_copy(s_sh.at[pl.ds(base_r + r0, B)], rows0)
            pltpu.sync_copy(rows0, s_out.at[cid, pl.ds(base_r + r0, B)])
        if with_deg:
            for t in range(RPT // B):
                r = base_r + t * B
                pltpu.sync_copy(deg_sh.at[pl.ds(r, B)], ones_v)
                pltpu.sync_copy(ones_v, deg_out.at[cid, pl.ds(r, B)])

    return pl.kernel(body, out_type=out_type, mesh=_SC_MESH,
                     scratch_types=scratch_types,
                     compiler_params=_SC_PARAMS)


_sc_layer_deg = _make_sc_layer(True)
_sc_layer = _make_sc_layer(False)


def _dot(a, b):
    return jnp.dot(a, b, preferred_element_type=_f32,
                   precision=lax.Precision.HIGHEST)


def _kx_body(x_ref, w_ref, o1_ref, o2_ref):
    y = _dot(x_ref[...], w_ref[...])
    o1_ref[...] = y[:, :H].astype(_bf16)
    o2_ref[...] = y[:, H:]


def _kea_body(ea_ref, w_ref, b_ref, o1_ref, o2_ref):
    y = _dot(ea_ref[...], w_ref[...])
    y = y + b_ref[...]
    o1_ref[...] = y[:, :H].astype(_bf16)
    o2_ref[...] = y[:, H:].astype(_bf16)


def _kmid_body(s_ref, deg_ref, xr_ref, g_ref, b_ref, w_ref, o1_ref, o2_ref):
    s = s_ref[0, :N, :].astype(_f32) + s_ref[1, :N, :].astype(_f32)
    deg = deg_ref[0, :N, 0:1].astype(_f32) + deg_ref[1, :N, 0:1].astype(_f32)
    h = xr_ref[...] + s / jnp.maximum(deg, 1.0)
    mu = jnp.mean(h, axis=0, keepdims=True)
    var = jnp.mean((h - mu) ** 2, axis=0, keepdims=True)
    h = (h - mu) * lax.rsqrt(var + EPS) * g_ref[...] + b_ref[...]
    h = jnp.maximum(h, 0.0)
    y = _dot(h, w_ref[...])
    o1_ref[...] = y[:, :H].astype(_bf16)
    o2_ref[...] = y[:, H:]


def _kfin_body(s_ref, deg_ref, xr_ref, g_ref, b_ref, wo_ref, bo_ref, o_ref):
    s = s_ref[0, :N, :].astype(_f32) + s_ref[1, :N, :].astype(_f32)
    deg = deg_ref[0, :N, 0:1].astype(_f32) + deg_ref[1, :N, 0:1].astype(_f32)
    h = xr_ref[...] + s / jnp.maximum(deg, 1.0)
    mu = jnp.mean(h, axis=0, keepdims=True)
    var = jnp.mean((h - mu) ** 2, axis=0, keepdims=True)
    h = (h - mu) * lax.rsqrt(var + EPS) * g_ref[...] + b_ref[...]
    h = jnp.maximum(h, 0.0)
    hg = jnp.mean(h, axis=0, keepdims=True)
    o_ref[...] = _dot(hg, wo_ref[...]) + bo_ref[...]


_EA_BLK = 2560


def _run_tc(x, Wcat1, ea, Wecat, bcat):
    xW1, xroot1 = pl.pallas_call(
        _kx_body,
        out_shape=[jax.ShapeDtypeStruct((N, H), _bf16),
                   jax.ShapeDtypeStruct((N, H), _f32)],
    )(x, Wcat1)
    eaW1, eaW2 = pl.pallas_call(
        _kea_body,
        grid=(E // _EA_BLK,),
        in_specs=[pl.BlockSpec((_EA_BLK, DE), lambda i: (i, 0)),
                  pl.BlockSpec((DE, 2 * H), lambda i: (0, 0)),
                  pl.BlockSpec((1, 2 * H), lambda i: (0, 0))],
        out_specs=[pl.BlockSpec((_EA_BLK, H), lambda i: (i, 0)),
                   pl.BlockSpec((_EA_BLK, H), lambda i: (i, 0))],
        out_shape=[jax.ShapeDtypeStruct((E, H), _bf16),
                   jax.ShapeDtypeStruct((E, H), _bf16)],
    )(ea, Wecat, bcat)
    return xW1, xroot1, eaW1, eaW2


def kernel(x, edge_index, edge_attr, W_neigh1, b_neigh1, W_root1, gamma1,
           beta1, W_neigh2, b_neigh2, W_root2, gamma2, beta2, W_out, b_out):
    srcm = edge_index[0].astype(jnp.int32).reshape(NROWS, B)
    dstm = edge_index[1].astype(jnp.int32).reshape(NROWS, B)

    Wcat1 = jnp.concatenate([W_neigh1[:D], W_root1], axis=1)
    Wecat = jnp.concatenate([W_neigh1[D:], W_neigh2[H:]], axis=1)
    bcat = jnp.concatenate([b_neigh1, b_neigh2]).reshape(1, 2 * H)
    Wcat2 = jnp.concatenate([W_neigh2[:H], W_root2], axis=1)

    xW1, xroot1, eaW1, eaW2 = _run_tc(x, Wcat1, edge_attr, Wecat, bcat)

    s1p, degp = _sc_layer_deg(xW1, eaW1, srcm, dstm)

    xW2, hroot2 = pl.pallas_call(
        _kmid_body,
        out_shape=[jax.ShapeDtypeStruct((N, H), _bf16),
                   jax.ShapeDtypeStruct((N, H), _f32)],
    )(s1p, degp, xroot1, gamma1.reshape(1, H), beta1.reshape(1, H), Wcat2)

    (s2p,) = _sc_layer(xW2, eaW2, srcm, dstm)

    out = pl.pallas_call(
        _kfin_body,
        out_shape=jax.ShapeDtypeStruct((1, 1), _f32),
    )(s2p, degp, hroot2, gamma2.reshape(1, H), beta2.reshape(1, H),
      W_out, b_out.reshape(1, 1))
    return out.reshape(1)


# speedup vs baseline: 5.6417x; 1.2596x over previous
import jax
import jax.numpy as jnp
from jax import lax
from jax.experimental import pallas as pl
from jax.experimental.pallas import tpu as pltpu
from jax.experimental.pallas import tpu_sc as plsc

N = 10000
E = 320000
D = 128
DE = 16
H = 64
EPS = 1e-5

NC = 2
NS = 16
NW = NC * NS
B = 128
NROWS = E // B
RPW_MAX = NROWS // NW + 1
PAIRS = NROWS // NW // 2
N_PAD = 10240
RPT = N_PAD // NS

_f32 = jnp.float32
_bf16 = jnp.bfloat16


_SC_MESH = plsc.VectorSubcoreMesh(core_axis_name="c", subcore_axis_name="s")
_SC_PARAMS = pltpu.CompilerParams(use_tc_tiling_on_sc=False)


def _make_sc_layer(with_deg: bool):
    out_type = [jax.ShapeDtypeStruct((NC, N_PAD, H), _bf16)]
    if with_deg:
        out_type.append(jax.ShapeDtypeStruct((NC, N_PAD, 16), _bf16))
    scratch_types = [
        pltpu.VMEM((RPW_MAX, B), jnp.int32),
        pltpu.VMEM((RPW_MAX, B), jnp.int32),
        pltpu.VMEM((B, H), _bf16),
        pltpu.VMEM((B, H), _bf16),
        pltpu.VMEM((B // 8, 8 * H), _bf16),
        pltpu.VMEM((B // 8, 8 * H), _bf16),
        pltpu.VMEM_SHARED((N_PAD, H), _bf16),
    ]
    if with_deg:
        scratch_types.append(pltpu.VMEM((B, 16), _bf16))
        scratch_types.append(pltpu.VMEM_SHARED((N_PAD, 16), _bf16))
    scratch_types += [pltpu.SemaphoreType.DMA] * 4

    def body(table, eaw, srcm, dstm, *rest):
        if with_deg:
            (s_out, deg_out, idx_s, idx_d, rows0, rows1, ea0, ea1, s_sh,
             ones_v, deg_sh, sg0, sg1, se0, se1) = rest
        else:
            (s_out, idx_s, idx_d, rows0, rows1, ea0, ea1, s_sh,
             sg0, sg1, se0, se1) = rest
            deg_out = ones_v = deg_sh = None
        rows = (rows0, rows1)
        ea = (ea0, ea1)
        sg = (sg0, sg1)
        se = (se0, se1)

        cid = lax.axis_index("c")
        sid = lax.axis_index("s")
        wid = sid * NC + cid
        lo = wid * NROWS // NW
        nr = (wid + 1) * NROWS // NW - lo
        zero32 = jnp.zeros((32,), _bf16)

        pltpu.sync_copy(srcm.at[pl.ds(lo, RPW_MAX)], idx_s)
        pltpu.sync_copy(dstm.at[pl.ds(lo, RPW_MAX)], idx_d)

        def zrow(j, _):
            for k in range(H // 32):
                rows0[j, pl.ds(k * 32, 32)] = zero32
            return 0
        lax.fori_loop(0, B, zrow, 0, unroll=4)

        base_r = sid * RPT
        for r0 in range(0, RPT, B):
            pltpu.sync_copy(rows0, s_sh.at[pl.ds(base_r + r0, B)])
        if with_deg:
            z216 = jnp.zeros((2, 16), _bf16)

            def zo(j, _):
                ones_v[pl.ds(2 * j, 2), :] = z216
                return 0
            lax.fori_loop(0, B // 2, zo, 0, unroll=4)
            for t in range(RPT // B):
                pltpu.sync_copy(ones_v, deg_sh.at[pl.ds(base_r + t * B, B)])
            o216 = jnp.full((2, 16), 1.0, _bf16)

            def oo(j, _):
                ones_v[pl.ds(2 * j, 2), :] = o216
                return 0
            lax.fori_loop(0, B // 2, oo, 0, unroll=4)

        plsc.subcore_barrier()

        def issue(r, p):
            pltpu.async_copy(eaw.at[pl.ds((lo + r) * (B // 8), B // 8)],
                             ea[p], se[p])
            pltpu.async_copy(table.at[idx_s.at[r]], rows[p], sg[p])

        def consume(r, p):
            pltpu.make_async_copy(table.at[pl.ds(0, B)], rows[p], sg[p]).wait()
            pltpu.make_async_copy(eaw.at[pl.ds(0, B // 8)], ea[p], se[p]).wait()

            def compute(j, _):
                for q in range(16):
                    sl32 = pl.ds(q * 32, 32)
                    sl = pl.ds((q % 2) * 32, 32)
                    jj = j * 8 + q // 2
                    rows[p][jj, sl] = jnp.maximum(
                        rows[p][jj, sl] + ea[p][j, sl32], 0.0)
                return 0
            lax.fori_loop(0, B // 8, compute, 0, unroll=2)

            pltpu.sync_copy(rows[p], s_sh.at[idx_d.at[r]], add=True)
            if with_deg:
                pltpu.sync_copy(ones_v, deg_sh.at[idx_d.at[r]], add=True)

        issue(0, 0)

        def step(t, _):
            r = 2 * t
            issue(r + 1, 1)
            consume(r, 0)

            @pl.when(r + 2 < nr)
            def _():
                issue(r + 2, 0)
            consume(r + 1, 1)
            return 0
        lax.fori_loop(0, PAIRS, step, 0)

        @pl.when(nr % 2 == 1)
        def _():
            consume(nr - 1, 0)
        plsc.subcore_barrier()

        for r0 in range(0, RPT, B):
            pltpu.sync_copy(s_sh.at[pl.ds(base_r + r0, B)], rows0)
            pltpu.sync_copy(rows0, s_out.at[cid, pl.ds(base_r + r0, B)])
        if with_deg:
            for t in range(RPT // B):
                r = base_r + t * B
                pltpu.sync_copy(deg_sh.at[pl.ds(r, B)], ones_v)
                pltpu.sync_copy(ones_v, deg_out.at[cid, pl.ds(r, B)])

    return pl.kernel(body, out_type=out_type, mesh=_SC_MESH,
                     scratch_types=scratch_types,
                     compiler_params=_SC_PARAMS)


_sc_layer_deg = _make_sc_layer(True)
_sc_layer = _make_sc_layer(False)


def _dot(a, b):
    return jnp.dot(a, b, preferred_element_type=_f32,
                   precision=lax.Precision.HIGHEST)


def _kx_body(x_ref, w_ref, o1_ref, o2_ref):
    y = _dot(x_ref[...], w_ref[...])
    o1_ref[...] = y[:, :H].astype(_bf16)
    o2_ref[...] = y[:, H:]


def _kea_body(ea_ref, w1_ref, w2_ref, b1_ref, b2_ref, o1_ref, o2_ref):
    a = ea_ref[...]
    o1_ref[...] = (jnp.dot(a, w1_ref[...], preferred_element_type=_f32)
                   + b1_ref[...]).astype(_bf16)
    o2_ref[...] = (jnp.dot(a, w2_ref[...], preferred_element_type=_f32)
                   + b2_ref[...]).astype(_bf16)


def _kmid_body(s_ref, deg_ref, xr_ref, g_ref, b_ref, w_ref, o1_ref, o2_ref):
    s = s_ref[0, :N, :].astype(_f32) + s_ref[1, :N, :].astype(_f32)
    deg = deg_ref[0, :N, 0:1].astype(_f32) + deg_ref[1, :N, 0:1].astype(_f32)
    h = xr_ref[...] + s / jnp.maximum(deg, 1.0)
    mu = jnp.mean(h, axis=0, keepdims=True)
    var = jnp.mean((h - mu) ** 2, axis=0, keepdims=True)
    h = (h - mu) * lax.rsqrt(var + EPS) * g_ref[...] + b_ref[...]
    h = jnp.maximum(h, 0.0)
    y = _dot(h, w_ref[...])
    o1_ref[...] = y[:, :H].astype(_bf16)
    o2_ref[...] = y[:, H:]


def _kfin_body(s_ref, deg_ref, xr_ref, g_ref, b_ref, wo_ref, bo_ref, o_ref):
    s = s_ref[0, :N, :].astype(_f32) + s_ref[1, :N, :].astype(_f32)
    deg = deg_ref[0, :N, 0:1].astype(_f32) + deg_ref[1, :N, 0:1].astype(_f32)
    h = xr_ref[...] + s / jnp.maximum(deg, 1.0)
    mu = jnp.mean(h, axis=0, keepdims=True)
    var = jnp.mean((h - mu) ** 2, axis=0, keepdims=True)
    h = (h - mu) * lax.rsqrt(var + EPS) * g_ref[...] + b_ref[...]
    h = jnp.maximum(h, 0.0)
    hg = jnp.mean(h, axis=0, keepdims=True)
    o_ref[...] = _dot(hg, wo_ref[...]) + bo_ref[...]


E8 = E // 8
_EA_BLK = 4000


def _run_tc(x, Wcat1, ea8, W81, W82, b81, b82):
    xW1, xroot1 = pl.pallas_call(
        _kx_body,
        out_shape=[jax.ShapeDtypeStruct((N, H), _bf16),
                   jax.ShapeDtypeStruct((N, H), _f32)],
    )(x, Wcat1)
    eaW1, eaW2 = pl.pallas_call(
        _kea_body,
        grid=(E8 // _EA_BLK,),
        in_specs=[pl.BlockSpec((_EA_BLK, D), lambda i: (i, 0)),
                  pl.BlockSpec((D, 8 * H), lambda i: (0, 0)),
                  pl.BlockSpec((D, 8 * H), lambda i: (0, 0)),
                  pl.BlockSpec((1, 8 * H), lambda i: (0, 0)),
                  pl.BlockSpec((1, 8 * H), lambda i: (0, 0))],
        out_specs=[pl.BlockSpec((_EA_BLK, 8 * H), lambda i: (i, 0)),
                   pl.BlockSpec((_EA_BLK, 8 * H), lambda i: (i, 0))],
        out_shape=[jax.ShapeDtypeStruct((E8, 8 * H), _bf16),
                   jax.ShapeDtypeStruct((E8, 8 * H), _bf16)],
    )(ea8, W81, W82, b81, b82)
    return xW1, xroot1, eaW1, eaW2


def kernel(x, edge_index, edge_attr, W_neigh1, b_neigh1, W_root1, gamma1,
           beta1, W_neigh2, b_neigh2, W_root2, gamma2, beta2, W_out, b_out):
    srcm = edge_index[0].astype(jnp.int32).reshape(NROWS, B)
    dstm = edge_index[1].astype(jnp.int32).reshape(NROWS, B)

    Wcat1 = jnp.concatenate([W_neigh1[:D], W_root1], axis=1)
    Wcat2 = jnp.concatenate([W_neigh2[:H], W_root2], axis=1)
    eye8 = jnp.eye(8, dtype=_f32)
    W81 = jnp.kron(eye8, W_neigh1[D:])
    W82 = jnp.kron(eye8, W_neigh2[H:])
    b81 = jnp.tile(b_neigh1, 8).reshape(1, 8 * H)
    b82 = jnp.tile(b_neigh2, 8).reshape(1, 8 * H)
    ea8 = edge_attr.reshape(E // 8, 8 * DE)

    xW1, xroot1, eaW1, eaW2 = _run_tc(x, Wcat1, ea8, W81, W82, b81, b82)

    s1p, degp = _sc_layer_deg(xW1, eaW1, srcm, dstm)

    xW2, hroot2 = pl.pallas_call(
        _kmid_body,
        out_shape=[jax.ShapeDtypeStruct((N, H), _bf16),
                   jax.ShapeDtypeStruct((N, H), _f32)],
    )(s1p, degp, xroot1, gamma1.reshape(1, H), beta1.reshape(1, H), Wcat2)

    (s2p,) = _sc_layer(xW2, eaW2, srcm, dstm)

    out = pl.pallas_call(
        _kfin_body,
        out_shape=jax.ShapeDtypeStruct((1, 1), _f32),
    )(s2p, degp, hroot2, gamma2.reshape(1, H), beta2.reshape(1, H),
      W_out, b_out.reshape(1, 1))
    return out.reshape(1)
